# Initial kernel scaffold; baseline (speedup 1.0000x reference)
import jax
import jax.numpy as jnp
from jax.experimental import pallas as pl


def kernel(x, edge_index, edge_type, W1_rel, W1_root, b1, W2_rel, W2_root, b2):
    raise NotImplementedError("write your pallas kernel here")


# speedup vs baseline: 13.2154x; 13.2154x over previous
import functools

import jax
import jax.numpy as jnp
from jax import lax
from jax.experimental import pallas as pl
from jax.experimental.pallas import tpu as pltpu
from jax.experimental.pallas import tpu_sc as plsc

N = 10000
E = 320000
R = 8
D_IN = 128
D_HID = 256
D_OUT = 128

L = 16
SEG = 128
NSEG = 16
BLK = SEG * NSEG
NTILES = 16
E_PAD = 327680
EPT = E_PAD // NTILES
NBLK = EPT // BLK
ROWS_E = E_PAD // SEG
RN = R * N
ACC_ROWS = 81920
APT = ACC_ROWS // NTILES
RPT = RN // NTILES


def _make_scatter(c_total, with_counts):
  cp = c_total // 2
  mesh = plsc.VectorSubcoreMesh(core_axis_name="c", subcore_axis_name="s")

  def body(*refs):
    if with_counts:
      (xT, srcm, dstm, zrows, zcnt, s4, cnt_o, acc, cnt_acc, idx_s, idx_d,
       upd, ones_v, sem_g, sem_s) = refs
    else:
      (xT, srcm, dstm, zrows, s4, acc, idx_s, idx_d, upd,
       sem_g, sem_s) = refs
    cid = lax.axis_index("c")
    sid = lax.axis_index("s")
    t5 = sid * RPT
    z0 = sid * APT

    if with_counts:
      def fill_ones(i, c):
        ones_v[pl.ds(i * L, L)] = jnp.ones((L,), jnp.float32)
        return c
      lax.fori_loop(0, BLK // L, fill_ones, 0)

      @pl.when(cid == 0)
      def _():
        pltpu.sync_copy(zcnt, cnt_acc.at[pl.ds(sid * (ACC_ROWS // NTILES),
                                               ACC_ROWS // NTILES)])

    for c_l in range(cp):
      c = cid * cp + c_l
      pltpu.sync_copy(zrows, acc.at[pl.ds(z0, APT // 2)])
      pltpu.sync_copy(zrows, acc.at[pl.ds(z0 + APT // 2, APT // 2)])
      plsc.subcore_barrier()

      def blk_body(b, carry):
        row0 = sid * (EPT // SEG) + b * NSEG
        pltpu.sync_copy(srcm.at[pl.ds(row0, NSEG)], idx_s)
        pltpu.sync_copy(dstm.at[pl.ds(row0, NSEG)], idx_d)
        gs = [pltpu.async_copy(xT.at[c].at[idx_s.at[j]],
                               upd.at[pl.ds(j * SEG, SEG)], sem_g)
              for j in range(NSEG)]
        for h in gs:
          h.wait()
        ss = [pltpu.async_copy(upd.at[pl.ds(j * SEG, SEG)],
                               acc.at[idx_d.at[j]], sem_s, add=True)
              for j in range(NSEG)]
        for h in ss:
          h.wait()
        if with_counts and c_l == 0:
          @pl.when(cid == 0)
          def _():
            cs = [pltpu.async_copy(ones_v.at[pl.ds(j * SEG, SEG)],
                                   cnt_acc.at[idx_d.at[j]], sem_s, add=True)
                  for j in range(NSEG)]
            for h in cs:
              h.wait()
        return carry

      lax.fori_loop(0, NBLK, blk_body, 0)
      plsc.subcore_barrier()
      pltpu.sync_copy(acc.at[pl.ds(t5, RPT)], s4.at[pl.ds(t5, RPT), c, :])
      if with_counts and c_l == 0:
        @pl.when(cid == 0)
        def _():
          pltpu.sync_copy(cnt_acc.at[pl.ds(t5, RPT)],
                          cnt_o.at[pl.ds(t5, RPT)])
      plsc.subcore_barrier()

  out_type = [jax.ShapeDtypeStruct((RN, c_total, L), jnp.float32)]
  scratch = [
      pltpu.VMEM_SHARED((ACC_ROWS, L), jnp.float32),
  ]
  if with_counts:
    out_type.append(jax.ShapeDtypeStruct((RN,), jnp.float32))
    scratch.append(pltpu.VMEM_SHARED((ACC_ROWS,), jnp.float32))
  scratch += [
      pltpu.VMEM((NSEG, SEG), jnp.int32),
      pltpu.VMEM((NSEG, SEG), jnp.int32),
      pltpu.VMEM((BLK, L), jnp.float32),
  ]
  if with_counts:
    scratch.append(pltpu.VMEM((BLK,), jnp.float32))
  scratch += [pltpu.SemaphoreType.DMA, pltpu.SemaphoreType.DMA]

  return pl.kernel(
      body, out_type=tuple(out_type), mesh=mesh,
      scratch_types=tuple(scratch),
      compiler_params=pltpu.CompilerParams(use_tc_tiling_on_sc=False))


def _tc_layer(x, s, cnt_t, w_root, w_rel, b, relu):
  n, d_in = x.shape
  d_out = w_root.shape[1]
  mb = 1000
  g = n // mb

  def tc_body(x_ref, s_ref, c_ref, wr_ref, wl_ref, b_ref, o_ref):
    inv = 1.0 / jnp.maximum(c_ref[...], 1.0)
    acc = jnp.dot(x_ref[...], wr_ref[...],
                  preferred_element_type=jnp.float32) + b_ref[...]
    for r in range(R):
      acc = acc + jnp.dot(s_ref[r] * inv[:, r:r + 1], wl_ref[r],
                          preferred_element_type=jnp.float32)
    o_ref[...] = jnp.maximum(acc, 0.0) if relu else acc

  return pl.pallas_call(
      tc_body,
      grid=(g,),
      in_specs=[
          pl.BlockSpec((mb, d_in), lambda i: (i, 0)),
          pl.BlockSpec((R, mb, d_in), lambda i: (0, i, 0)),
          pl.BlockSpec((mb, R), lambda i: (i, 0)),
          pl.BlockSpec((d_in, d_out), lambda i: (0, 0)),
          pl.BlockSpec((R, d_in, d_out), lambda i: (0, 0, 0)),
          pl.BlockSpec((1, d_out), lambda i: (0, 0)),
      ],
      out_specs=pl.BlockSpec((mb, d_out), lambda i: (i, 0)),
      out_shape=jax.ShapeDtypeStruct((n, d_out), jnp.float32),
  )(x, s, cnt_t, w_root, w_rel, b.reshape(1, -1))


def kernel(x, edge_index, edge_type, W1_rel, W1_root, b1, W2_rel, W2_root,
           b2):
  src = edge_index[0].astype(jnp.int32)
  dst = edge_index[1].astype(jnp.int32)
  npad = E_PAD - E
  pad_src = jnp.arange(npad, dtype=jnp.int32) % N
  pad_dump = RN + jnp.arange(npad, dtype=jnp.int32) % (ACC_ROWS - RN)
  sidx = jnp.concatenate(
      [edge_type.astype(jnp.int32) * N + dst, pad_dump]).reshape(ROWS_E, SEG)
  srcm = jnp.concatenate([src, pad_src]).reshape(ROWS_E, SEG)

  zrows = jnp.zeros((APT // 2, L), jnp.float32)
  zcnt = jnp.zeros((ACC_ROWS // NTILES,), jnp.float32)
  xT = x.reshape(N, D_IN // L, L).transpose(1, 0, 2)
  s4, cnt = _make_scatter(D_IN // L, True)(xT, srcm, sidx, zrows, zcnt)
  s1 = s4.reshape(R, N, D_IN)
  cnt_t = cnt.reshape(R, N).T

  h = _tc_layer(x, s1, cnt_t, W1_root, W1_rel, b1, relu=True)

  hT = h.reshape(N, D_HID // L, L).transpose(1, 0, 2)
  s4b, = _make_scatter(D_HID // L, False)(hT, srcm, sidx, zrows)
  s2 = s4b.reshape(R, N, D_HID)

  return _tc_layer(h, s2, cnt_t, W2_root, W2_rel, b2, relu=False)


# speedup vs baseline: 15.0565x; 1.1393x over previous
import functools

import jax
import jax.numpy as jnp
from jax import lax
from jax.experimental import pallas as pl
from jax.experimental.pallas import tpu as pltpu
from jax.experimental.pallas import tpu_sc as plsc

N = 10000
E = 320000
R = 8
D_IN = 128
D_HID = 256
D_OUT = 128

L = 16
SEG = 128
NSEG = 16
BLK = SEG * NSEG
NTILES = 16
E_PAD = 327680
EPT = E_PAD // NTILES
NBLK = EPT // BLK
ROWS_E = E_PAD // SEG
RN = R * N
ACC_ROWS = 81920
APT = ACC_ROWS // NTILES
RPT = RN // NTILES
ZROWS = 256


def _make_scatter(c_total, with_counts):
  cp = c_total // 2
  mesh = plsc.VectorSubcoreMesh(core_axis_name="c", subcore_axis_name="s")

  def body(*refs):
    if with_counts:
      (xT, srcm, dstm, zcnt, s4, cnt_o, acc, cnt_acc, idx_s, idx_d,
       upd, ones_v, zbuf, sem_i, sem_g, sem_s) = refs
    else:
      (xT, srcm, dstm, s4, acc, idx_s, idx_d, upd, zbuf,
       sem_i, sem_g, sem_s) = refs
    cid = lax.axis_index("c")
    sid = lax.axis_index("s")
    t5 = sid * RPT
    z0 = sid * APT

    def fill_zbuf(i, c):
      zbuf[i, :] = jnp.zeros((L,), jnp.float32)
      return c
    lax.fori_loop(0, ZROWS, fill_zbuf, 0)

    if with_counts:
      def fill_ones(i, c):
        ones_v[pl.ds(i * L, L)] = jnp.ones((L,), jnp.float32)
        return c
      lax.fori_loop(0, SEG // L, fill_ones, 0)

      @pl.when(cid == 0)
      def _():
        pltpu.sync_copy(zcnt, cnt_acc.at[pl.ds(sid * (ACC_ROWS // NTILES),
                                               ACC_ROWS // NTILES)])

    for c_l in range(cp):
      c = cid * cp + c_l
      def zero_blk(k, carry):
        pltpu.sync_copy(zbuf, acc.at[pl.ds(z0 + k * ZROWS, ZROWS)])
        return carry
      lax.fori_loop(0, APT // ZROWS, zero_blk, 0)
      plsc.subcore_barrier()

      def blk_body(b, carry):
        row0 = sid * (EPT // SEG) + b * NSEG
        i1 = pltpu.async_copy(srcm.at[pl.ds(row0, NSEG)], idx_s, sem_i)
        i2 = pltpu.async_copy(dstm.at[pl.ds(row0, NSEG)], idx_d, sem_i)
        i1.wait()
        i2.wait()
        gs = [pltpu.async_copy(xT.at[c].at[idx_s.at[j]],
                               upd.at[pl.ds(j * SEG, SEG)], sem_g)
              for j in range(NSEG)]
        h = NSEG // 2
        for j in range(h):
          gs[j].wait()
        ss = [pltpu.async_copy(upd.at[pl.ds(j * SEG, SEG)],
                               acc.at[idx_d.at[j]], sem_s, add=True)
              for j in range(h)]
        for j in range(h, NSEG):
          gs[j].wait()
        ss += [pltpu.async_copy(upd.at[pl.ds(j * SEG, SEG)],
                                acc.at[idx_d.at[j]], sem_s, add=True)
               for j in range(h, NSEG)]
        if with_counts and c_l == 0:
          @pl.when(cid == 0)
          def _():
            cs = [pltpu.async_copy(ones_v, cnt_acc.at[idx_d.at[j]], sem_s,
                                   add=True)
                  for j in range(NSEG)]
            for hh in cs:
              hh.wait()
        for hh in ss:
          hh.wait()
        return carry

      lax.fori_loop(0, NBLK, blk_body, 0)
      plsc.subcore_barrier()
      pltpu.sync_copy(acc.at[pl.ds(t5, RPT)], s4.at[pl.ds(t5, RPT), c, :])
      if with_counts and c_l == 0:
        @pl.when(cid == 0)
        def _():
          pltpu.sync_copy(cnt_acc.at[pl.ds(t5, RPT)],
                          cnt_o.at[pl.ds(t5, RPT)])
      plsc.subcore_barrier()

  out_type = [jax.ShapeDtypeStruct((RN, c_total, L), jnp.float32)]
  scratch = [
      pltpu.VMEM_SHARED((ACC_ROWS, L), jnp.float32),
  ]
  if with_counts:
    out_type.append(jax.ShapeDtypeStruct((RN,), jnp.float32))
    scratch.append(pltpu.VMEM_SHARED((ACC_ROWS,), jnp.float32))
  scratch += [
      pltpu.VMEM((NSEG, SEG), jnp.int32),
      pltpu.VMEM((NSEG, SEG), jnp.int32),
      pltpu.VMEM((BLK, L), jnp.float32),
  ]
  if with_counts:
    scratch.append(pltpu.VMEM((SEG,), jnp.float32))
  scratch.append(pltpu.VMEM((ZROWS, L), jnp.float32))
  scratch += [pltpu.SemaphoreType.DMA, pltpu.SemaphoreType.DMA,
              pltpu.SemaphoreType.DMA]

  return pl.kernel(
      body, out_type=tuple(out_type), mesh=mesh,
      scratch_types=tuple(scratch),
      compiler_params=pltpu.CompilerParams(use_tc_tiling_on_sc=False))


def _tc_layer(x, s, cnt_t, w_root, w_rel, b, relu):
  n, d_in = x.shape
  d_out = w_root.shape[1]
  mb = 1000
  g = n // mb

  def tc_body(x_ref, s_ref, c_ref, wr_ref, wl_ref, b_ref, o_ref):
    inv = 1.0 / jnp.maximum(c_ref[...], 1.0)
    acc = jnp.dot(x_ref[...], wr_ref[...],
                  preferred_element_type=jnp.float32) + b_ref[...]
    for r in range(R):
      acc = acc + jnp.dot(s_ref[r] * inv[:, r:r + 1], wl_ref[r],
                          preferred_element_type=jnp.float32)
    o_ref[...] = jnp.maximum(acc, 0.0) if relu else acc

  return pl.pallas_call(
      tc_body,
      grid=(g,),
      in_specs=[
          pl.BlockSpec((mb, d_in), lambda i: (i, 0)),
          pl.BlockSpec((R, mb, d_in), lambda i: (0, i, 0)),
          pl.BlockSpec((mb, R), lambda i: (i, 0)),
          pl.BlockSpec((d_in, d_out), lambda i: (0, 0)),
          pl.BlockSpec((R, d_in, d_out), lambda i: (0, 0, 0)),
          pl.BlockSpec((1, d_out), lambda i: (0, 0)),
      ],
      out_specs=pl.BlockSpec((mb, d_out), lambda i: (i, 0)),
      out_shape=jax.ShapeDtypeStruct((n, d_out), jnp.float32),
  )(x, s, cnt_t, w_root, w_rel, b.reshape(1, -1))


def kernel(x, edge_index, edge_type, W1_rel, W1_root, b1, W2_rel, W2_root,
           b2):
  src = edge_index[0].astype(jnp.int32)
  dst = edge_index[1].astype(jnp.int32)
  npad = E_PAD - E
  pad_src = jnp.arange(npad, dtype=jnp.int32) % N
  pad_dump = RN + jnp.arange(npad, dtype=jnp.int32) % (ACC_ROWS - RN)
  sidx = jnp.concatenate(
      [edge_type.astype(jnp.int32) * N + dst, pad_dump]).reshape(ROWS_E, SEG)
  srcm = jnp.concatenate([src, pad_src]).reshape(ROWS_E, SEG)

  zcnt = jnp.zeros((ACC_ROWS // NTILES,), jnp.float32)
  xT = x.reshape(N, D_IN // L, L).transpose(1, 0, 2)
  s4, cnt = _make_scatter(D_IN // L, True)(xT, srcm, sidx, zcnt)
  s1 = s4.reshape(R, N, D_IN)
  cnt_t = cnt.reshape(R, N).T

  h = _tc_layer(x, s1, cnt_t, W1_root, W1_rel, b1, relu=True)

  hT = h.reshape(N, D_HID // L, L).transpose(1, 0, 2)
  s4b, = _make_scatter(D_HID // L, False)(hT, srcm, sidx)
  s2 = s4b.reshape(R, N, D_HID)

  return _tc_layer(h, s2, cnt_t, W2_root, W2_rel, b2, relu=False)


# speedup vs baseline: 15.3745x; 1.0211x over previous
import functools

import jax
import jax.numpy as jnp
from jax import lax
from jax.experimental import pallas as pl
from jax.experimental.pallas import tpu as pltpu
from jax.experimental.pallas import tpu_sc as plsc

N = 10000
E = 320000
R = 8
D_IN = 128
D_HID = 256
D_OUT = 128

L = 16
SEG = 128
NSEG = 10
BLK = SEG * NSEG
NTILES = 16
E_PAD = 327680
EPT = E_PAD // NTILES
NBLK = EPT // BLK
ROWS_E = E_PAD // SEG
RN = R * N
ACC_ROWS = 81920
APT = ACC_ROWS // NTILES
RPT = RN // NTILES
ZROWS = 128


def _make_scatter(c_total, with_counts):
  cp = c_total // 2
  mesh = plsc.VectorSubcoreMesh(core_axis_name="c", subcore_axis_name="s")

  def body(*refs):
    if with_counts:
      (xT, srcm, dstm, zcnt, s4, cnt_o, acc, cnt_acc, tab, idx_s, idx_d,
       upd, ones_v, zbuf, sem_i, sem_g, sem_s) = refs
    else:
      (xT, srcm, dstm, s4, acc, tab, idx_s, idx_d, upd, zbuf,
       sem_i, sem_g, sem_s) = refs
    cid = lax.axis_index("c")
    sid = lax.axis_index("s")
    t5 = sid * RPT
    z0 = sid * APT

    def fill_zbuf(i, c):
      zbuf[i, :] = jnp.zeros((L,), jnp.float32)
      return c
    lax.fori_loop(0, ZROWS, fill_zbuf, 0)

    if with_counts:
      def fill_ones(i, c):
        ones_v[pl.ds(i * L, L)] = jnp.ones((L,), jnp.float32)
        return c
      lax.fori_loop(0, SEG // L, fill_ones, 0)

      @pl.when(cid == 0)
      def _():
        pltpu.sync_copy(zcnt, cnt_acc.at[pl.ds(sid * (ACC_ROWS // NTILES),
                                               ACC_ROWS // NTILES)])

    for c_l in range(cp):
      c = cid * cp + c_l
      pltpu.sync_copy(xT.at[c, pl.ds(sid * (N // NTILES), N // NTILES)],
                      tab.at[pl.ds(sid * (N // NTILES), N // NTILES)])
      def zero_blk(k, carry):
        pltpu.sync_copy(zbuf, acc.at[pl.ds(z0 + k * ZROWS, ZROWS)])
        return carry
      lax.fori_loop(0, APT // ZROWS, zero_blk, 0)
      plsc.subcore_barrier()

      def blk_body(b, carry):
        row0 = sid * (EPT // SEG) + b * NSEG
        i1 = pltpu.async_copy(srcm.at[pl.ds(row0, NSEG)], idx_s, sem_i)
        i2 = pltpu.async_copy(dstm.at[pl.ds(row0, NSEG)], idx_d, sem_i)
        i1.wait()
        i2.wait()
        gs = [pltpu.async_copy(tab.at[idx_s.at[j]],
                               upd.at[pl.ds(j * SEG, SEG)], sem_g)
              for j in range(NSEG)]
        h = NSEG // 2
        for j in range(h):
          gs[j].wait()
        ss = [pltpu.async_copy(upd.at[pl.ds(j * SEG, SEG)],
                               acc.at[idx_d.at[j]], sem_s, add=True)
              for j in range(h)]
        for j in range(h, NSEG):
          gs[j].wait()
        ss += [pltpu.async_copy(upd.at[pl.ds(j * SEG, SEG)],
                                acc.at[idx_d.at[j]], sem_s, add=True)
               for j in range(h, NSEG)]
        if with_counts and c_l == 0:
          @pl.when(cid == 0)
          def _():
            cs = [pltpu.async_copy(ones_v, cnt_acc.at[idx_d.at[j]], sem_s,
                                   add=True)
                  for j in range(NSEG)]
            for hh in cs:
              hh.wait()
        for hh in ss:
          hh.wait()
        return carry

      lax.fori_loop(0, NBLK, blk_body, 0)
      plsc.subcore_barrier()
      pltpu.sync_copy(acc.at[pl.ds(t5, RPT)], s4.at[pl.ds(t5, RPT), c, :])
      if with_counts and c_l == 0:
        @pl.when(cid == 0)
        def _():
          pltpu.sync_copy(cnt_acc.at[pl.ds(t5, RPT)],
                          cnt_o.at[pl.ds(t5, RPT)])
      plsc.subcore_barrier()

  out_type = [jax.ShapeDtypeStruct((RN, c_total, L), jnp.float32)]
  scratch = [
      pltpu.VMEM_SHARED((ACC_ROWS, L), jnp.float32),
  ]
  if with_counts:
    out_type.append(jax.ShapeDtypeStruct((RN,), jnp.float32))
    scratch.append(pltpu.VMEM_SHARED((ACC_ROWS,), jnp.float32))
  scratch += [
      pltpu.VMEM_SHARED((N, L), jnp.float32),
      pltpu.VMEM((NSEG, SEG), jnp.int32),
      pltpu.VMEM((NSEG, SEG), jnp.int32),
      pltpu.VMEM((BLK, L), jnp.float32),
  ]
  if with_counts:
    scratch.append(pltpu.VMEM((SEG,), jnp.float32))
  scratch.append(pltpu.VMEM((ZROWS, L), jnp.float32))
  scratch += [pltpu.SemaphoreType.DMA, pltpu.SemaphoreType.DMA,
              pltpu.SemaphoreType.DMA]

  return pl.kernel(
      body, out_type=tuple(out_type), mesh=mesh,
      scratch_types=tuple(scratch),
      compiler_params=pltpu.CompilerParams(use_tc_tiling_on_sc=False))


def _tc_layer(x, s, cnt_t, w_root, w_rel, b, relu):
  n, d_in = x.shape
  d_out = w_root.shape[1]
  mb = 1000
  g = n // mb

  def tc_body(x_ref, s_ref, c_ref, wr_ref, wl_ref, b_ref, o_ref):
    inv = 1.0 / jnp.maximum(c_ref[...], 1.0)
    acc = jnp.dot(x_ref[...], wr_ref[...],
                  preferred_element_type=jnp.float32) + b_ref[...]
    for r in range(R):
      acc = acc + jnp.dot(s_ref[r] * inv[:, r:r + 1], wl_ref[r],
                          preferred_element_type=jnp.float32)
    o_ref[...] = jnp.maximum(acc, 0.0) if relu else acc

  return pl.pallas_call(
      tc_body,
      grid=(g,),
      in_specs=[
          pl.BlockSpec((mb, d_in), lambda i: (i, 0)),
          pl.BlockSpec((R, mb, d_in), lambda i: (0, i, 0)),
          pl.BlockSpec((mb, R), lambda i: (i, 0)),
          pl.BlockSpec((d_in, d_out), lambda i: (0, 0)),
          pl.BlockSpec((R, d_in, d_out), lambda i: (0, 0, 0)),
          pl.BlockSpec((1, d_out), lambda i: (0, 0)),
      ],
      out_specs=pl.BlockSpec((mb, d_out), lambda i: (i, 0)),
      out_shape=jax.ShapeDtypeStruct((n, d_out), jnp.float32),
  )(x, s, cnt_t, w_root, w_rel, b.reshape(1, -1))


def kernel(x, edge_index, edge_type, W1_rel, W1_root, b1, W2_rel, W2_root,
           b2):
  src = edge_index[0].astype(jnp.int32)
  dst = edge_index[1].astype(jnp.int32)
  npad = E_PAD - E
  pad_src = jnp.arange(npad, dtype=jnp.int32) % N
  pad_dump = RN + jnp.arange(npad, dtype=jnp.int32) % (ACC_ROWS - RN)
  sidx = jnp.concatenate(
      [edge_type.astype(jnp.int32) * N + dst, pad_dump]).reshape(ROWS_E, SEG)
  srcm = jnp.concatenate([src, pad_src]).reshape(ROWS_E, SEG)

  zcnt = jnp.zeros((ACC_ROWS // NTILES,), jnp.float32)
  xT = x.reshape(N, D_IN // L, L).transpose(1, 0, 2)
  s4, cnt = _make_scatter(D_IN // L, True)(xT, srcm, sidx, zcnt)
  s1 = s4.reshape(R, N, D_IN)
  cnt_t = cnt.reshape(R, N).T

  h = _tc_layer(x, s1, cnt_t, W1_root, W1_rel, b1, relu=True)

  hT = h.reshape(N, D_HID // L, L).transpose(1, 0, 2)
  s4b, = _make_scatter(D_HID // L, False)(hT, srcm, sidx)
  s2 = s4b.reshape(R, N, D_HID)

  return _tc_layer(h, s2, cnt_t, W2_root, W2_rel, b2, relu=False)


# speedup vs baseline: 15.6127x; 1.0155x over previous
import jax
import jax.numpy as jnp
from jax import lax
from jax.experimental import pallas as pl
from jax.experimental.pallas import tpu as pltpu
from jax.experimental.pallas import tpu_sc as plsc

N = 10000
E = 320000
R = 8
D_IN = 128
D_HID = 256
D_OUT = 128

L = 16
SEG = 128
NSEG = 8
BLK = SEG * NSEG
NTILES = 16
E_PAD = 327680
EPT = E_PAD // NTILES
NBLK = EPT // BLK
ROWS_E = E_PAD // SEG
RN = R * N
ACC_ROWS = 81920
APT = ACC_ROWS // NTILES
RPT = RN // NTILES
ZROWS = 128


def _make_scatter(c_total, with_counts):
  cp = c_total // 2
  mesh = plsc.VectorSubcoreMesh(core_axis_name="c", subcore_axis_name="s")

  def body(*refs):
    if with_counts:
      (xT, srcm, dstm, zcnt, s4, cnt_o, acc, cnt_acc, idx_s, idx_d,
       upd, ones_v, zbuf, sem_i, sem_g, sem_s) = refs
    else:
      (xT, srcm, dstm, s4, acc, idx_s, idx_d, upd, zbuf,
       sem_i, sem_g, sem_s) = refs
    cid = lax.axis_index("c")
    sid = lax.axis_index("s")
    t5 = sid * RPT
    z0 = sid * APT

    def fill_zbuf(i, c):
      zbuf[i, :] = jnp.zeros((L,), jnp.float32)
      return c
    lax.fori_loop(0, ZROWS, fill_zbuf, 0)

    if with_counts:
      def fill_ones(i, c):
        ones_v[pl.ds(i * L, L)] = jnp.ones((L,), jnp.float32)
        return c
      lax.fori_loop(0, SEG // L, fill_ones, 0)

      @pl.when(cid == 0)
      def _():
        pltpu.sync_copy(zcnt, cnt_acc.at[pl.ds(sid * (ACC_ROWS // NTILES),
                                               ACC_ROWS // NTILES)])

    def idx_rows(b):
      return sid * (EPT // SEG) + b * NSEG

    def fire_idx(b, par):
      pltpu.async_copy(srcm.at[pl.ds(idx_rows(b), NSEG)], idx_s.at[par],
                       sem_i)
      pltpu.async_copy(dstm.at[pl.ds(idx_rows(b), NSEG)], idx_d.at[par],
                       sem_i)

    def drain_idx(b, par):
      pltpu.make_async_copy(srcm.at[pl.ds(idx_rows(b), NSEG)],
                            idx_s.at[par], sem_i).wait()
      pltpu.make_async_copy(dstm.at[pl.ds(idx_rows(b), NSEG)],
                            idx_d.at[par], sem_i).wait()

    def drain_scatters(par, counts):
      for j in range(NSEG):
        pltpu.make_async_copy(upd.at[par, pl.ds(j * SEG, SEG)],
                              acc.at[idx_d.at[par, j]], sem_s).wait()
      if counts:
        @pl.when(cid == 0)
        def _():
          for j in range(NSEG):
            pltpu.make_async_copy(ones_v, cnt_acc.at[idx_d.at[par, j]],
                                  sem_s).wait()

    for c_l in range(cp):
      c = cid * cp + c_l
      counts_here = with_counts and c_l == 0
      def zero_blk(k, carry):
        pltpu.sync_copy(zbuf, acc.at[pl.ds(z0 + k * ZROWS, ZROWS)])
        return carry
      lax.fori_loop(0, APT // ZROWS, zero_blk, 0)
      plsc.subcore_barrier()

      fire_idx(0, 0)

      def blk_body(b, carry):
        par = lax.rem(b, 2)
        alt = 1 - par
        drain_idx(b, par)
        gs = [pltpu.async_copy(xT.at[c].at[idx_s.at[par, j]],
                               upd.at[par, pl.ds(j * SEG, SEG)], sem_g)
              for j in range(NSEG)]

        @pl.when(b > 0)
        def _():
          drain_scatters(alt, counts_here)

        @pl.when(b + 1 < NBLK)
        def _():
          fire_idx(b + 1, alt)

        for h in gs:
          h.wait()
        for j in range(NSEG):
          pltpu.async_copy(upd.at[par, pl.ds(j * SEG, SEG)],
                           acc.at[idx_d.at[par, j]], sem_s, add=True)
        if counts_here:
          @pl.when(cid == 0)
          def _():
            for j in range(NSEG):
              pltpu.async_copy(ones_v, cnt_acc.at[idx_d.at[par, j]],
                               sem_s, add=True)
        return carry

      lax.fori_loop(0, NBLK, blk_body, 0)
      drain_scatters((NBLK - 1) % 2, counts_here)
      plsc.subcore_barrier()
      pltpu.sync_copy(acc.at[pl.ds(t5, RPT)], s4.at[pl.ds(t5, RPT), c, :])
      if counts_here:
        @pl.when(cid == 0)
        def _():
          pltpu.sync_copy(cnt_acc.at[pl.ds(t5, RPT)],
                          cnt_o.at[pl.ds(t5, RPT)])
      plsc.subcore_barrier()

  out_type = [jax.ShapeDtypeStruct((RN, c_total, L), jnp.float32)]
  scratch = [
      pltpu.VMEM_SHARED((ACC_ROWS, L), jnp.float32),
  ]
  if with_counts:
    out_type.append(jax.ShapeDtypeStruct((RN,), jnp.float32))
    scratch.append(pltpu.VMEM_SHARED((ACC_ROWS,), jnp.float32))
  scratch += [
      pltpu.VMEM((2, NSEG, SEG), jnp.int32),
      pltpu.VMEM((2, NSEG, SEG), jnp.int32),
      pltpu.VMEM((2, BLK, L), jnp.float32),
  ]
  if with_counts:
    scratch.append(pltpu.VMEM((SEG,), jnp.float32))
  scratch.append(pltpu.VMEM((ZROWS, L), jnp.float32))
  scratch += [pltpu.SemaphoreType.DMA, pltpu.SemaphoreType.DMA,
              pltpu.SemaphoreType.DMA]

  return pl.kernel(
      body, out_type=tuple(out_type), mesh=mesh,
      scratch_types=tuple(scratch),
      compiler_params=pltpu.CompilerParams(use_tc_tiling_on_sc=False))


def _tc_layer(x, s, cnt_t, w_root, w_rel, b, relu):
  n, d_in = x.shape
  d_out = w_root.shape[1]
  mb = 1000
  g = n // mb

  def tc_body(x_ref, s_ref, c_ref, wr_ref, wl_ref, b_ref, o_ref):
    inv = 1.0 / jnp.maximum(c_ref[...], 1.0)
    acc = jnp.dot(x_ref[...], wr_ref[...],
                  preferred_element_type=jnp.float32) + b_ref[...]
    for r in range(R):
      acc = acc + jnp.dot(s_ref[r] * inv[:, r:r + 1], wl_ref[r],
                          preferred_element_type=jnp.float32)
    o_ref[...] = jnp.maximum(acc, 0.0) if relu else acc

  return pl.pallas_call(
      tc_body,
      grid=(g,),
      in_specs=[
          pl.BlockSpec((mb, d_in), lambda i: (i, 0)),
          pl.BlockSpec((R, mb, d_in), lambda i: (0, i, 0)),
          pl.BlockSpec((mb, R), lambda i: (i, 0)),
          pl.BlockSpec((d_in, d_out), lambda i: (0, 0)),
          pl.BlockSpec((R, d_in, d_out), lambda i: (0, 0, 0)),
          pl.BlockSpec((1, d_out), lambda i: (0, 0)),
      ],
      out_specs=pl.BlockSpec((mb, d_out), lambda i: (i, 0)),
      out_shape=jax.ShapeDtypeStruct((n, d_out), jnp.float32),
  )(x, s, cnt_t, w_root, w_rel, b.reshape(1, -1))


def kernel(x, edge_index, edge_type, W1_rel, W1_root, b1, W2_rel, W2_root,
           b2):
  src = edge_index[0].astype(jnp.int32)
  dst = edge_index[1].astype(jnp.int32)
  npad = E_PAD - E
  pad_src = jnp.arange(npad, dtype=jnp.int32) % N
  pad_dump = RN + jnp.arange(npad, dtype=jnp.int32) % (ACC_ROWS - RN)
  sidx = jnp.concatenate(
      [edge_type.astype(jnp.int32) * N + dst, pad_dump]).reshape(ROWS_E, SEG)
  srcm = jnp.concatenate([src, pad_src]).reshape(ROWS_E, SEG)

  zcnt = jnp.zeros((ACC_ROWS // NTILES,), jnp.float32)
  xT = x.reshape(N, D_IN // L, L).transpose(1, 0, 2)
  s4, cnt = _make_scatter(D_IN // L, True)(xT, srcm, sidx, zcnt)
  s1 = s4.reshape(R, N, D_IN)
  cnt_t = cnt.reshape(R, N).T

  h = _tc_layer(x, s1, cnt_t, W1_root, W1_rel, b1, relu=True)

  hT = h.reshape(N, D_HID // L, L).transpose(1, 0, 2)
  s4b, = _make_scatter(D_HID // L, False)(hT, srcm, sidx)
  s2 = s4b.reshape(R, N, D_HID)

  return _tc_layer(h, s2, cnt_t, W2_root, W2_rel, b2, relu=False)
